# Initial kernel scaffold; baseline (speedup 1.0000x reference)
import jax
import jax.numpy as jnp
from jax.experimental import pallas as pl


def kernel(edge_index, dist, stops, weekday, vehicles, emb, W_l, b_l, W_r, W_e, b_e, W_c, b_c):
    raise NotImplementedError("write your pallas kernel here")


# speedup vs baseline: 85.4096x; 85.4096x over previous
import jax
import jax.numpy as jnp
from jax.experimental import pallas as pl
from jax.experimental.pallas import tpu as pltpu

N = 1024
EMB = 12
E = 16
P = 32
S = 512
R1 = 128
NB1 = N // R1
R2 = 256
NB2 = N // R2

_HI = jax.lax.Precision.HIGHEST


def _dot(a, b):
    return jax.lax.dot_general(a, b, (((1,), (0,)), ((), ())),
                               precision=_HI,
                               preferred_element_type=jnp.float32)


def _phase_a_kernel(emb_ref, embT_ref, stops_r_ref, stops_c_ref,
                    W_l_ref, W_r_ref, W_lT_ref, W_rT_ref,
                    b_l_row_ref, b_l_col_ref, We_l_ref, We_rT_ref,
                    pref_ref, u_ref, act_ref, v_ref,
                    aggrT_acc, colsum_acc):
    i = pl.program_id(0)
    emb = emb_ref[...]
    embT = embT_ref[...]
    stops_r = stops_r_ref[...]
    stops_c = stops_c_ref[...]

    emb_blk = emb_ref[pl.ds(i * R1, R1), :]
    norm_col = jnp.sqrt(jnp.sum(emb_blk * emb_blk, axis=1, keepdims=True))
    xn_blk = emb_blk / jnp.maximum(norm_col, 1e-8)
    norm_row = jnp.sqrt(jnp.sum(embT * embT, axis=0, keepdims=True))
    xnT = embT / jnp.maximum(norm_row, 1e-8)

    sim = _dot(xn_blk, xnT)

    row_g = i * R1 + jax.lax.broadcasted_iota(jnp.int32, (R1, 1), 0)
    col_g = jax.lax.broadcasted_iota(jnp.int32, (1, N), 1)
    col_iota = jax.lax.broadcasted_iota(jnp.int32, (S, N), 1)
    act_row = jnp.any(col_iota == stops_c, axis=0, keepdims=True)
    act_col = jnp.any(row_g == stops_r, axis=1, keepdims=True)
    valid = (row_g != col_g) & act_col & act_row
    A = jnp.where(valid & (sim > 0.5), 1.0, 0.0)

    deg_col = jnp.maximum(jnp.sum(A, axis=1, keepdims=True), 1.0)
    aggr = _dot(A, emb) / deg_col
    pref = (_dot(aggr, W_l_ref[...]) + _dot(emb_blk, W_r_ref[...])
            + b_l_row_ref[...])
    pref_ref[...] = pref
    u_ref[...] = _dot(pref, We_l_ref[...])
    act_ref[...] = act_col.astype(jnp.float32)

    @pl.when(i == 0)
    def _init():
        aggrT_acc[...] = jnp.zeros_like(aggrT_acc)
        colsum_acc[...] = jnp.zeros_like(colsum_acc)

    embT_blk = embT_ref[:, pl.ds(i * R1, R1)]
    aggrT_acc[...] += _dot(embT_blk, A)
    colsum_acc[...] += jnp.sum(A, axis=0, keepdims=True)

    @pl.when(i == NB1 - 1)
    def _finish():
        deg_row = jnp.maximum(colsum_acc[...], 1.0)
        aggrT = aggrT_acc[...] / deg_row
        prefT = (_dot(W_lT_ref[...], aggrT) + _dot(W_rT_ref[...], embT)
                 + b_l_col_ref[...])
        v_ref[...] = _dot(We_rT_ref[...], prefT)


def _phase_b_kernel(pref_ref, u_ref, act_ref, v_ref, dist_ref,
                    Wc_p_ref, Wc_es_ref, Wc_d_ref, Wc3_ref,
                    bc_ref, scal_ref, out_ref):
    u = u_ref[...]
    v = v_ref[...]
    wk = scal_ref[0:1, 0:1]
    vh = scal_ref[0:1, 1:2]
    be = scal_ref[0:1, 2:3]
    es = u + v + be
    es = jnp.where(es > 0, es, 0.01 * es)
    acc = _dot(es, Wc_es_ref[...])
    acc += _dot(dist_ref[...], Wc_d_ref[...])
    acc += _dot(pref_ref[...], Wc_p_ref[...])
    acc += bc_ref[...] + wk * Wc3_ref[0:1, :] + vh * Wc3_ref[1:2, :]
    acc += act_ref[...] * Wc3_ref[2:3, :]
    m = jnp.max(acc, axis=1, keepdims=True)
    sh = acc - m
    lse = jnp.log(jnp.sum(jnp.exp(sh), axis=1, keepdims=True))
    out_ref[...] = sh - lse


def kernel(edge_index, dist, stops, weekday, vehicles, emb,
           W_l, b_l, W_r, W_e, b_e, W_c, b_c):
    del edge_index
    f32 = jnp.float32
    emb = emb.astype(f32)
    emb16 = jnp.pad(emb, ((0, 0), (0, E - EMB)))
    embT16 = emb16.T
    stops_r = stops.reshape(1, S)
    stops_c = stops.reshape(S, 1)
    W_l16 = jnp.pad(W_l.astype(f32), ((0, E - EMB), (0, 0)))
    W_r16 = jnp.pad(W_r.astype(f32), ((0, E - EMB), (0, 0)))
    W_lT16 = W_l16.T
    W_rT16 = W_r16.T
    b_l_row = b_l.reshape(1, P).astype(f32)
    b_l_col = b_l.reshape(P, 1).astype(f32)
    We_l = W_e[:P].astype(f32)
    We_rT = W_e[P:].reshape(1, P).astype(f32)

    const_spec = lambda shape: pl.BlockSpec(shape, lambda i: (0, 0))

    pref, u, act, v = pl.pallas_call(
        _phase_a_kernel,
        grid=(NB1,),
        in_specs=[
            const_spec((N, E)), const_spec((E, N)),
            const_spec((1, S)), const_spec((S, 1)),
            const_spec((E, P)), const_spec((E, P)),
            const_spec((P, E)), const_spec((P, E)),
            const_spec((1, P)), const_spec((P, 1)),
            const_spec((P, 1)), const_spec((1, P)),
        ],
        out_specs=[
            pl.BlockSpec((R1, P), lambda i: (i, 0)),
            pl.BlockSpec((R1, 1), lambda i: (i, 0)),
            pl.BlockSpec((R1, 1), lambda i: (i, 0)),
            pl.BlockSpec((1, N), lambda i: (0, 0)),
        ],
        out_shape=[
            jax.ShapeDtypeStruct((N, P), f32),
            jax.ShapeDtypeStruct((N, 1), f32),
            jax.ShapeDtypeStruct((N, 1), f32),
            jax.ShapeDtypeStruct((1, N), f32),
        ],
        scratch_shapes=[
            pltpu.VMEM((E, N), f32),
            pltpu.VMEM((1, N), f32),
        ],
    )(emb16, embT16, stops_r, stops_c, W_l16, W_r16, W_lT16, W_rT16,
      b_l_row, b_l_col, We_l, We_rT)

    Wc = W_c.astype(f32)
    Wc_p = Wc[:P]
    Wc_es = Wc[P:P + N]
    Wc_d = Wc[P + N:P + 2 * N]
    Wc3 = Wc[P + 2 * N:]
    bc_row = b_c.reshape(1, N).astype(f32)
    scal = jnp.stack([
        jnp.asarray(weekday, f32).reshape(()),
        jnp.asarray(vehicles, f32).reshape(()),
        b_e.reshape(()).astype(f32),
        jnp.float32(0.0),
    ]).reshape(1, 4)

    out = pl.pallas_call(
        _phase_b_kernel,
        grid=(NB2,),
        in_specs=[
            pl.BlockSpec((R2, P), lambda i: (i, 0)),
            pl.BlockSpec((R2, 1), lambda i: (i, 0)),
            pl.BlockSpec((R2, 1), lambda i: (i, 0)),
            const_spec((1, N)),
            pl.BlockSpec((R2, N), lambda i: (i, 0)),
            const_spec((P, N)),
            const_spec((N, N)),
            const_spec((N, N)),
            const_spec((3, N)),
            const_spec((1, N)),
            const_spec((1, 4)),
        ],
        out_specs=pl.BlockSpec((R2, N), lambda i: (i, 0)),
        out_shape=jax.ShapeDtypeStruct((N, N), f32),
    )(pref, u, act, v, dist.astype(f32), Wc_p, Wc_es, Wc_d, Wc3,
      bc_row, scal)
    return out


# speedup vs baseline: 124.1835x; 1.4540x over previous
import jax
import jax.numpy as jnp
from jax.experimental import pallas as pl
from jax.experimental.pallas import tpu as pltpu

N = 1024
EMB = 12
E = 16
P = 32
S = 512
R1 = 128
NB1 = N // R1
R2 = 256
NB2 = N // R2

_HI = jax.lax.Precision.HIGHEST


def _dot(a, b):
    return jax.lax.dot_general(a, b, (((1,), (0,)), ((), ())),
                               precision=_HI,
                               preferred_element_type=jnp.float32)


def _dot_fast(a, b):
    return jax.lax.dot_general(a, b, (((1,), (0,)), ((), ())),
                               precision=jax.lax.Precision.DEFAULT,
                               preferred_element_type=jnp.float32)


def _phase_a_kernel(emb_ref, embT_ref, stops_r_ref, stops_c_ref,
                    W_l_ref, W_r_ref, W_lT_ref, W_rT_ref,
                    b_l_row_ref, b_l_col_ref, We_l_ref, We_rT_ref,
                    pref_ref, u_ref, act_ref, v_ref,
                    aggrT_acc, colsum_acc):
    i = pl.program_id(0)
    emb = emb_ref[...]
    embT = embT_ref[...]
    stops_r = stops_r_ref[...]
    stops_c = stops_c_ref[...]

    emb_blk = emb_ref[pl.ds(i * R1, R1), :]
    norm_col = jnp.sqrt(jnp.sum(emb_blk * emb_blk, axis=1, keepdims=True))
    xn_blk = emb_blk / jnp.maximum(norm_col, 1e-8)
    norm_row = jnp.sqrt(jnp.sum(embT * embT, axis=0, keepdims=True))
    xnT = embT / jnp.maximum(norm_row, 1e-8)

    sim = _dot(xn_blk, xnT)

    row_g = i * R1 + jax.lax.broadcasted_iota(jnp.int32, (R1, 1), 0)
    col_g = jax.lax.broadcasted_iota(jnp.int32, (1, N), 1)
    col_iota = jax.lax.broadcasted_iota(jnp.int32, (S, N), 1)
    act_row = jnp.any(col_iota == stops_c, axis=0, keepdims=True)
    act_col = jnp.any(row_g == stops_r, axis=1, keepdims=True)
    valid = (row_g != col_g) & act_col & act_row
    A = jnp.where(valid & (sim > 0.5), 1.0, 0.0)

    deg_col = jnp.maximum(jnp.sum(A, axis=1, keepdims=True), 1.0)
    aggr = _dot(A, emb) / deg_col
    pref = (_dot(aggr, W_l_ref[...]) + _dot(emb_blk, W_r_ref[...])
            + b_l_row_ref[...])
    pref_ref[...] = pref
    u_ref[...] = _dot(pref, We_l_ref[...])
    act_ref[...] = act_col.astype(jnp.float32)

    @pl.when(i == 0)
    def _init():
        aggrT_acc[...] = jnp.zeros_like(aggrT_acc)
        colsum_acc[...] = jnp.zeros_like(colsum_acc)

    embT_blk = embT_ref[:, pl.ds(i * R1, R1)]
    aggrT_acc[...] += _dot(embT_blk, A)
    colsum_acc[...] += jnp.sum(A, axis=0, keepdims=True)

    @pl.when(i == NB1 - 1)
    def _finish():
        deg_row = jnp.maximum(colsum_acc[...], 1.0)
        aggrT = aggrT_acc[...] / deg_row
        prefT = (_dot(W_lT_ref[...], aggrT) + _dot(W_rT_ref[...], embT)
                 + b_l_col_ref[...])
        v_ref[...] = _dot(We_rT_ref[...], prefT)


def _phase_b_kernel(pref_ref, u_ref, act_ref, v_ref, dist_ref,
                    Wc_p_ref, Wc_es_ref, Wc_d_ref, Wc3_ref,
                    bc_ref, scal_ref, out_ref):
    u = u_ref[...]
    v = v_ref[...]
    wk = scal_ref[0:1, 0:1]
    vh = scal_ref[0:1, 1:2]
    be = scal_ref[0:1, 2:3]
    es = u + v + be
    es = jnp.where(es > 0, es, 0.01 * es)
    acc = _dot_fast(es, Wc_es_ref[...])
    acc += _dot_fast(dist_ref[...], Wc_d_ref[...])
    acc += _dot_fast(pref_ref[...], Wc_p_ref[...])
    acc += bc_ref[...] + wk * Wc3_ref[0:1, :] + vh * Wc3_ref[1:2, :]
    acc += act_ref[...] * Wc3_ref[2:3, :]
    m = jnp.max(acc, axis=1, keepdims=True)
    sh = acc - m
    lse = jnp.log(jnp.sum(jnp.exp(sh), axis=1, keepdims=True))
    out_ref[...] = sh - lse


def kernel(edge_index, dist, stops, weekday, vehicles, emb,
           W_l, b_l, W_r, W_e, b_e, W_c, b_c):
    del edge_index
    f32 = jnp.float32
    emb = emb.astype(f32)
    emb16 = jnp.pad(emb, ((0, 0), (0, E - EMB)))
    embT16 = emb16.T
    stops_r = stops.reshape(1, S)
    stops_c = stops.reshape(S, 1)
    W_l16 = jnp.pad(W_l.astype(f32), ((0, E - EMB), (0, 0)))
    W_r16 = jnp.pad(W_r.astype(f32), ((0, E - EMB), (0, 0)))
    W_lT16 = W_l16.T
    W_rT16 = W_r16.T
    b_l_row = b_l.reshape(1, P).astype(f32)
    b_l_col = b_l.reshape(P, 1).astype(f32)
    We_l = W_e[:P].astype(f32)
    We_rT = W_e[P:].reshape(1, P).astype(f32)

    const_spec = lambda shape: pl.BlockSpec(shape, lambda i: (0, 0))

    pref, u, act, v = pl.pallas_call(
        _phase_a_kernel,
        grid=(NB1,),
        in_specs=[
            const_spec((N, E)), const_spec((E, N)),
            const_spec((1, S)), const_spec((S, 1)),
            const_spec((E, P)), const_spec((E, P)),
            const_spec((P, E)), const_spec((P, E)),
            const_spec((1, P)), const_spec((P, 1)),
            const_spec((P, 1)), const_spec((1, P)),
        ],
        out_specs=[
            pl.BlockSpec((R1, P), lambda i: (i, 0)),
            pl.BlockSpec((R1, 1), lambda i: (i, 0)),
            pl.BlockSpec((R1, 1), lambda i: (i, 0)),
            pl.BlockSpec((1, N), lambda i: (0, 0)),
        ],
        out_shape=[
            jax.ShapeDtypeStruct((N, P), f32),
            jax.ShapeDtypeStruct((N, 1), f32),
            jax.ShapeDtypeStruct((N, 1), f32),
            jax.ShapeDtypeStruct((1, N), f32),
        ],
        scratch_shapes=[
            pltpu.VMEM((E, N), f32),
            pltpu.VMEM((1, N), f32),
        ],
    )(emb16, embT16, stops_r, stops_c, W_l16, W_r16, W_lT16, W_rT16,
      b_l_row, b_l_col, We_l, We_rT)

    Wc = W_c.astype(f32)
    Wc_p = Wc[:P]
    Wc_es = Wc[P:P + N]
    Wc_d = Wc[P + N:P + 2 * N]
    Wc3 = Wc[P + 2 * N:]
    bc_row = b_c.reshape(1, N).astype(f32)
    scal = jnp.stack([
        jnp.asarray(weekday, f32).reshape(()),
        jnp.asarray(vehicles, f32).reshape(()),
        b_e.reshape(()).astype(f32),
        jnp.float32(0.0),
    ]).reshape(1, 4)

    out = pl.pallas_call(
        _phase_b_kernel,
        grid=(NB2,),
        in_specs=[
            pl.BlockSpec((R2, P), lambda i: (i, 0)),
            pl.BlockSpec((R2, 1), lambda i: (i, 0)),
            pl.BlockSpec((R2, 1), lambda i: (i, 0)),
            const_spec((1, N)),
            pl.BlockSpec((R2, N), lambda i: (i, 0)),
            const_spec((P, N)),
            const_spec((N, N)),
            const_spec((N, N)),
            const_spec((3, N)),
            const_spec((1, N)),
            const_spec((1, 4)),
        ],
        out_specs=pl.BlockSpec((R2, N), lambda i: (i, 0)),
        out_shape=jax.ShapeDtypeStruct((N, N), f32),
    )(pref, u, act, v, dist.astype(f32), Wc_p, Wc_es, Wc_d, Wc3,
      bc_row, scal)
    return out


# speedup vs baseline: 191.7441x; 1.5440x over previous
import jax
import jax.numpy as jnp
from jax.experimental import pallas as pl
from jax.experimental.pallas import tpu as pltpu

N = 1024
EMB = 12
E = 16
P = 32
S = 512
R1 = 256
NB1 = N // R1
R2 = 256
NB2 = N // R2

_HI = jax.lax.Precision.HIGHEST
_DEF = jax.lax.Precision.DEFAULT


def _dot(a, b, prec=_HI):
    return jax.lax.dot_general(a, b, (((1,), (0,)), ((), ())),
                               precision=prec,
                               preferred_element_type=jnp.float32)


def _dot_fast(a, b):
    return _dot(a, b, _DEF)


def _phase_a_kernel(emb_ref, embT_ref, stops_r_ref, stops_c_ref,
                    W_l_ref, W_r_ref, W_lT_ref, W_rT_ref,
                    b_l_row_ref, b_l_col_ref, We_l_ref, We_rT_ref,
                    pref_ref, u_ref, act_ref, v_ref,
                    aggrT_acc, colsum_acc, xnT_s, act_row_s):
    i = pl.program_id(0)
    embT = embT_ref[...]

    @pl.when(i == 0)
    def _init():
        norm_row = jnp.sqrt(jnp.sum(embT * embT, axis=0, keepdims=True))
        xnT_s[...] = embT / jnp.maximum(norm_row, 1e-8)
        col_iota = jax.lax.broadcasted_iota(jnp.int32, (S, N), 1)
        hit = (col_iota == stops_c_ref[...]).astype(jnp.float32)
        act_row_s[...] = jnp.max(hit, axis=0, keepdims=True)
        aggrT_acc[...] = jnp.zeros_like(aggrT_acc)
        colsum_acc[...] = jnp.zeros_like(colsum_acc)

    emb_blk = emb_ref[pl.ds(i * R1, R1), :]
    norm_col = jnp.sqrt(jnp.sum(emb_blk * emb_blk, axis=1, keepdims=True))
    xn_blk = emb_blk / jnp.maximum(norm_col, 1e-8)

    sim = _dot(xn_blk, xnT_s[...])

    row_g = i * R1 + jax.lax.broadcasted_iota(jnp.int32, (R1, 1), 0)
    col_g = jax.lax.broadcasted_iota(jnp.int32, (1, N), 1)
    act_col = jnp.any(row_g == stops_r_ref[...], axis=1, keepdims=True)
    valid = (row_g != col_g) & act_col & (act_row_s[...] > 0.0)
    A = jnp.where(valid & (sim > 0.5), 1.0, 0.0)

    deg_col = jnp.maximum(jnp.sum(A, axis=1, keepdims=True), 1.0)
    aggr = _dot_fast(A, emb_ref[...]) / deg_col
    pref = (_dot_fast(aggr, W_l_ref[...]) + _dot_fast(emb_blk, W_r_ref[...])
            + b_l_row_ref[...])
    pref_ref[...] = pref
    u_ref[...] = _dot_fast(pref, We_l_ref[...])
    act_ref[...] = act_col.astype(jnp.float32)

    embT_blk = embT_ref[:, pl.ds(i * R1, R1)]
    aggrT_acc[...] += _dot_fast(embT_blk, A)
    colsum_acc[...] += jnp.sum(A, axis=0, keepdims=True)

    @pl.when(i == NB1 - 1)
    def _finish():
        deg_row = jnp.maximum(colsum_acc[...], 1.0)
        aggrT = aggrT_acc[...] / deg_row
        prefT = (_dot_fast(W_lT_ref[...], aggrT)
                 + _dot_fast(W_rT_ref[...], embT)
                 + b_l_col_ref[...])
        v_ref[...] = _dot_fast(We_rT_ref[...], prefT)


def _phase_b_kernel(pref_ref, u_ref, act_ref, v_ref, dist_ref,
                    Wc_ref, bc_ref, scal_ref, out_ref):
    u = u_ref[...]
    v = v_ref[...]
    wk = scal_ref[0:1, 0:1]
    vh = scal_ref[0:1, 1:2]
    be = scal_ref[0:1, 2:3]
    es = u + v + be
    es = jnp.where(es > 0, es, 0.01 * es)
    acc = _dot_fast(es, Wc_ref[P:P + N, :])
    acc += _dot_fast(dist_ref[...], Wc_ref[P + N:P + 2 * N, :])
    acc += _dot_fast(pref_ref[...], Wc_ref[0:P, :])
    acc += (bc_ref[...] + wk * Wc_ref[P + 2 * N:P + 2 * N + 1, :]
            + vh * Wc_ref[P + 2 * N + 1:P + 2 * N + 2, :])
    acc += act_ref[...] * Wc_ref[P + 2 * N + 2:P + 2 * N + 3, :]
    m = jnp.max(acc, axis=1, keepdims=True)
    sh = acc - m
    lse = jnp.log(jnp.sum(jnp.exp(sh), axis=1, keepdims=True))
    out_ref[...] = sh - lse


def kernel(edge_index, dist, stops, weekday, vehicles, emb,
           W_l, b_l, W_r, W_e, b_e, W_c, b_c):
    del edge_index
    f32 = jnp.float32
    emb = emb.astype(f32)
    emb16 = jnp.pad(emb, ((0, 0), (0, E - EMB)))
    embT16 = emb16.T
    stops_r = stops.reshape(1, S)
    stops_c = stops.reshape(S, 1)
    W_l16 = jnp.pad(W_l.astype(f32), ((0, E - EMB), (0, 0)))
    W_r16 = jnp.pad(W_r.astype(f32), ((0, E - EMB), (0, 0)))
    W_lT16 = W_l16.T
    W_rT16 = W_r16.T
    b_l_row = b_l.reshape(1, P).astype(f32)
    b_l_col = b_l.reshape(P, 1).astype(f32)
    We_l = W_e[:P].astype(f32)
    We_rT = W_e[P:].reshape(1, P).astype(f32)

    const_spec = lambda shape: pl.BlockSpec(shape, lambda i: (0, 0))

    pref, u, act, v = pl.pallas_call(
        _phase_a_kernel,
        grid=(NB1,),
        in_specs=[
            const_spec((N, E)), const_spec((E, N)),
            const_spec((1, S)), const_spec((S, 1)),
            const_spec((E, P)), const_spec((E, P)),
            const_spec((P, E)), const_spec((P, E)),
            const_spec((1, P)), const_spec((P, 1)),
            const_spec((P, 1)), const_spec((1, P)),
        ],
        out_specs=[
            pl.BlockSpec((R1, P), lambda i: (i, 0)),
            pl.BlockSpec((R1, 1), lambda i: (i, 0)),
            pl.BlockSpec((R1, 1), lambda i: (i, 0)),
            pl.BlockSpec((1, N), lambda i: (0, 0)),
        ],
        out_shape=[
            jax.ShapeDtypeStruct((N, P), f32),
            jax.ShapeDtypeStruct((N, 1), f32),
            jax.ShapeDtypeStruct((N, 1), f32),
            jax.ShapeDtypeStruct((1, N), f32),
        ],
        scratch_shapes=[
            pltpu.VMEM((E, N), f32),
            pltpu.VMEM((1, N), f32),
            pltpu.VMEM((E, N), f32),
            pltpu.VMEM((1, N), f32),
        ],
    )(emb16, embT16, stops_r, stops_c, W_l16, W_r16, W_lT16, W_rT16,
      b_l_row, b_l_col, We_l, We_rT)

    bc_row = b_c.reshape(1, N).astype(f32)
    scal = jnp.stack([
        jnp.asarray(weekday, f32).reshape(()),
        jnp.asarray(vehicles, f32).reshape(()),
        b_e.reshape(()).astype(f32),
        jnp.float32(0.0),
    ]).reshape(1, 4)

    out = pl.pallas_call(
        _phase_b_kernel,
        grid=(NB2,),
        in_specs=[
            pl.BlockSpec((R2, P), lambda i: (i, 0)),
            pl.BlockSpec((R2, 1), lambda i: (i, 0)),
            pl.BlockSpec((R2, 1), lambda i: (i, 0)),
            const_spec((1, N)),
            pl.BlockSpec((R2, N), lambda i: (i, 0)),
            const_spec((P + 2 * N + 3, N)),
            const_spec((1, N)),
            const_spec((1, 4)),
        ],
        out_specs=pl.BlockSpec((R2, N), lambda i: (i, 0)),
        out_shape=jax.ShapeDtypeStruct((N, N), f32),
    )(pref, u, act, v, dist.astype(f32), W_c.astype(f32), bc_row, scal)
    return out


# speedup vs baseline: 192.5807x; 1.0044x over previous
import jax
import jax.numpy as jnp
from jax.experimental import pallas as pl
from jax.experimental.pallas import tpu as pltpu

N = 1024
EMB = 12
E = 16
P = 32
S = 512
R = 256
NB = N // R

_HI = jax.lax.Precision.HIGHEST
_DEF = jax.lax.Precision.DEFAULT


def _dot(a, b, prec=_HI):
    return jax.lax.dot_general(a, b, (((1,), (0,)), ((), ())),
                               precision=prec,
                               preferred_element_type=jnp.float32)


def _dot_fast(a, b):
    return _dot(a, b, _DEF)


def _fused_kernel(emb_ref, embT_ref, stops_r_ref, stops_c_ref,
                  W_l_ref, W_r_ref, b_l_row_ref, We_l_ref,
                  gl_ref, gr_ref, dist_ref, Wc_ref, bc_ref, scal_ref,
                  out_ref,
                  pref_s, u_s, actc_s, v_s,
                  aggrT_acc, colsum_acc, xnT_s, act_row_s):
    i = pl.program_id(0)

    @pl.when(i == 0)
    def _init():
        embT = embT_ref[...]
        norm_row = jnp.sqrt(jnp.sum(embT * embT, axis=0, keepdims=True))
        xnT_s[...] = embT / jnp.maximum(norm_row, 1e-8)
        col_iota = jax.lax.broadcasted_iota(jnp.int32, (S, N), 1)
        hit = (col_iota == stops_c_ref[...]).astype(jnp.float32)
        act_row_s[...] = jnp.max(hit, axis=0, keepdims=True)
        aggrT_acc[...] = jnp.zeros_like(aggrT_acc)
        colsum_acc[...] = jnp.zeros_like(colsum_acc)

    @pl.when(i < NB)
    def _phase_a():
        emb_blk = emb_ref[pl.ds(i * R, R), :]
        norm_col = jnp.sqrt(jnp.sum(emb_blk * emb_blk, axis=1, keepdims=True))
        xn_blk = emb_blk / jnp.maximum(norm_col, 1e-8)

        sim = _dot(xn_blk, xnT_s[...])

        row_g = i * R + jax.lax.broadcasted_iota(jnp.int32, (R, 1), 0)
        col_g = jax.lax.broadcasted_iota(jnp.int32, (1, N), 1)
        act_col = jnp.any(row_g == stops_r_ref[...], axis=1, keepdims=True)
        valid = (row_g != col_g) & act_col & (act_row_s[...] > 0.0)
        A = jnp.where(valid & (sim > 0.5), 1.0, 0.0)

        deg_col = jnp.maximum(jnp.sum(A, axis=1, keepdims=True), 1.0)
        aggr = _dot_fast(A, emb_ref[...]) / deg_col
        pref = (_dot_fast(aggr, W_l_ref[...])
                + _dot_fast(emb_blk, W_r_ref[...])
                + b_l_row_ref[...])
        pref_s[pl.ds(i * R, R), :] = pref
        u_s[pl.ds(i * R, R), :] = _dot_fast(pref, We_l_ref[...])
        actc_s[pl.ds(i * R, R), :] = act_col.astype(jnp.float32)

        embT_blk = embT_ref[:, pl.ds(i * R, R)]
        aggrT_acc[...] += _dot_fast(embT_blk, A)
        colsum_acc[...] += jnp.sum(A, axis=0, keepdims=True)

        @pl.when(i == NB - 1)
        def _finish_a():
            deg_row = jnp.maximum(colsum_acc[...], 1.0)
            aggrT = aggrT_acc[...] / deg_row
            c0 = scal_ref[0:1, 3:4]
            v_s[...] = (_dot_fast(gl_ref[...], aggrT)
                        + _dot_fast(gr_ref[...], embT_ref[...]) + c0)

    @pl.when(i >= NB)
    def _phase_b():
        j = i - NB
        u = u_s[pl.ds(j * R, R), :]
        v = v_s[...]
        wk = scal_ref[0:1, 0:1]
        vh = scal_ref[0:1, 1:2]
        be = scal_ref[0:1, 2:3]
        es = u + v + be
        es = jnp.where(es > 0, es, 0.01 * es)
        acc = _dot_fast(es, Wc_ref[P:P + N, :])
        acc += _dot_fast(dist_ref[...], Wc_ref[P + N:P + 2 * N, :])
        acc += _dot_fast(pref_s[pl.ds(j * R, R), :], Wc_ref[0:P, :])
        acc += (bc_ref[...] + wk * Wc_ref[P + 2 * N:P + 2 * N + 1, :]
                + vh * Wc_ref[P + 2 * N + 1:P + 2 * N + 2, :])
        acc += actc_s[pl.ds(j * R, R), :] * Wc_ref[P + 2 * N + 2:P + 2 * N + 3, :]
        m = jnp.max(acc, axis=1, keepdims=True)
        sh = acc - m
        lse = jnp.log(jnp.sum(jnp.exp(sh), axis=1, keepdims=True))
        out_ref[...] = sh - lse


def kernel(edge_index, dist, stops, weekday, vehicles, emb,
           W_l, b_l, W_r, W_e, b_e, W_c, b_c):
    del edge_index
    f32 = jnp.float32
    emb = emb.astype(f32)
    emb16 = jnp.pad(emb, ((0, 0), (0, E - EMB)))
    embT16 = emb16.T
    stops_r = stops.reshape(1, S)
    stops_c = stops.reshape(S, 1)
    W_l16 = jnp.pad(W_l.astype(f32), ((0, E - EMB), (0, 0)))
    W_r16 = jnp.pad(W_r.astype(f32), ((0, E - EMB), (0, 0)))
    b_l_row = b_l.reshape(1, P).astype(f32)
    We_l = W_e[:P].astype(f32)
    We_r = W_e[P:].astype(f32)
    gl = (W_l16 @ We_r).reshape(1, E)
    gr = (W_r16 @ We_r).reshape(1, E)
    c0 = (b_l @ We_r).reshape(())
    bc_row = b_c.reshape(1, N).astype(f32)
    scal = jnp.stack([
        jnp.asarray(weekday, f32).reshape(()),
        jnp.asarray(vehicles, f32).reshape(()),
        b_e.reshape(()).astype(f32),
        c0.astype(f32),
    ]).reshape(1, 4)

    const_spec = lambda shape: pl.BlockSpec(shape, lambda i: (0, 0))

    out = pl.pallas_call(
        _fused_kernel,
        grid=(2 * NB,),
        in_specs=[
            const_spec((N, E)), const_spec((E, N)),
            const_spec((1, S)), const_spec((S, 1)),
            const_spec((E, P)), const_spec((E, P)),
            const_spec((1, P)), const_spec((P, 1)),
            const_spec((1, E)), const_spec((1, E)),
            pl.BlockSpec((R, N), lambda i: (jnp.maximum(i - NB, 0), 0)),
            const_spec((P + 2 * N + 3, N)),
            const_spec((1, N)),
            const_spec((1, 4)),
        ],
        out_specs=pl.BlockSpec((R, N), lambda i: (jnp.maximum(i - NB, 0), 0)),
        out_shape=jax.ShapeDtypeStruct((N, N), f32),
        scratch_shapes=[
            pltpu.VMEM((N, P), f32),
            pltpu.VMEM((N, 1), f32),
            pltpu.VMEM((N, 1), f32),
            pltpu.VMEM((1, N), f32),
            pltpu.VMEM((E, N), f32),
            pltpu.VMEM((1, N), f32),
            pltpu.VMEM((E, N), f32),
            pltpu.VMEM((1, N), f32),
        ],
    )(emb16, embT16, stops_r, stops_c, W_l16, W_r16, b_l_row, We_l,
      gl, gr, dist.astype(f32), W_c.astype(f32), bc_row, scal)
    return out


# speedup vs baseline: 292.9282x; 1.5211x over previous
import jax
import jax.numpy as jnp
from jax.experimental import pallas as pl
from jax.experimental.pallas import tpu as pltpu

N = 1024
EMB = 12
P = 32
S = 512
R = 256
NB = N // R

_HI = jax.lax.Precision.HIGHEST
_DEF = jax.lax.Precision.DEFAULT


def _dot(a, b, prec=_DEF):
    return jax.lax.dot_general(a, b, (((1,), (0,)), ((), ())),
                               precision=prec,
                               preferred_element_type=jnp.float32)


def _dot_nt(a, b, prec=_DEF):
    return jax.lax.dot_general(a, b, (((1,), (1,)), ((), ())),
                               precision=prec,
                               preferred_element_type=jnp.float32)


def _fused_kernel(emb_ref, stops_r_ref, W_l_ref, W_r_ref, b_l_row_ref,
                  W_eT_ref, dist_ref, Wc_ref, bc_ref, be_ref, wv_ref,
                  out_ref,
                  xn_s, pref_s, u_s, act_s, v_s,
                  vacc_s, h_s, hr_s, colsum_s, act_row_s):
    i = pl.program_id(0)

    @pl.when(i == 0)
    def _init():
        emb = emb_ref[...]
        norm = jnp.sqrt(jnp.sum(emb * emb, axis=1, keepdims=True))
        xn_s[...] = emb / jnp.maximum(norm, 1e-8)
        row_iota = jax.lax.broadcasted_iota(jnp.int32, (N, 1), 0)
        oh = (row_iota == stops_r_ref[...]).astype(jnp.float32)
        act_s[...] = jnp.max(oh, axis=1, keepdims=True)
        ones_row = jnp.ones((1, S), jnp.float32)
        act_row_s[...] = jnp.minimum(_dot_nt(ones_row, oh), 1.0)
        We_r_row = W_eT_ref[:, P:]
        gl = _dot_nt(We_r_row, W_l_ref[...])
        gr = _dot_nt(We_r_row, W_r_ref[...])
        h_s[...] = _dot_nt(gl, emb)
        hr_s[...] = _dot_nt(gr, emb)
        vacc_s[...] = jnp.zeros_like(vacc_s)
        colsum_s[...] = jnp.zeros_like(colsum_s)

    @pl.when(i < NB)
    def _phase_a():
        emb_blk = emb_ref[pl.ds(i * R, R), :]
        xn_blk = xn_s[pl.ds(i * R, R), :]

        sim = _dot_nt(xn_blk, xn_s[...], _HI)

        row_g = i * R + jax.lax.broadcasted_iota(jnp.int32, (R, 1), 0)
        col_g = jax.lax.broadcasted_iota(jnp.int32, (1, N), 1)
        act_col = act_s[pl.ds(i * R, R), :] > 0.0
        valid = (row_g != col_g) & act_col & (act_row_s[...] > 0.0)
        A = jnp.where(valid & (sim > 0.5), 1.0, 0.0)

        deg_col = jnp.maximum(jnp.sum(A, axis=1, keepdims=True), 1.0)
        aggr = _dot(A, emb_ref[...]) / deg_col
        pref = (_dot(aggr, W_l_ref[...]) + _dot(emb_blk, W_r_ref[...])
                + b_l_row_ref[...])
        pref_s[pl.ds(i * R, R), :] = pref
        u_s[pl.ds(i * R, R), :] = _dot_nt(pref, W_eT_ref[:, :P])

        vacc_s[...] += _dot(h_s[:, pl.ds(i * R, R)], A)
        colsum_s[...] += jnp.sum(A, axis=0, keepdims=True)

        @pl.when(i == NB - 1)
        def _finish_a():
            deg_row = jnp.maximum(colsum_s[...], 1.0)
            c0 = jnp.sum(b_l_row_ref[...] * W_eT_ref[:, P:],
                         axis=1, keepdims=True)
            v_s[...] = vacc_s[...] / deg_row + hr_s[...] + c0

    @pl.when(i >= NB)
    def _phase_b():
        j = i - NB
        u = u_s[pl.ds(j * R, R), :]
        v = v_s[...]
        es = u + v + be_ref[...]
        es = jnp.where(es > 0, es, 0.01 * es)
        acc = _dot(es, Wc_ref[P:P + N, :])
        acc += _dot(dist_ref[...], Wc_ref[P + N:P + 2 * N, :])
        acc += _dot(pref_s[pl.ds(j * R, R), :], Wc_ref[0:P, :])
        acc += (bc_ref[...] + wv_ref[0:1, 0:1] * Wc_ref[P + 2 * N:P + 2 * N + 1, :]
                + wv_ref[0:1, 1:2] * Wc_ref[P + 2 * N + 1:P + 2 * N + 2, :])
        acc += act_s[pl.ds(j * R, R), :] * Wc_ref[P + 2 * N + 2:P + 2 * N + 3, :]
        m = jnp.max(acc, axis=1, keepdims=True)
        sh = acc - m
        lse = jnp.log(jnp.sum(jnp.exp(sh), axis=1, keepdims=True))
        out_ref[...] = sh - lse


def kernel(edge_index, dist, stops, weekday, vehicles, emb,
           W_l, b_l, W_r, W_e, b_e, W_c, b_c):
    del edge_index
    f32 = jnp.float32
    stops_r = stops.reshape(1, S)
    W_eT = W_e.reshape(1, 2 * P).astype(f32)
    b_l_row = b_l.reshape(1, P).astype(f32)
    bc_row = b_c.reshape(1, N).astype(f32)
    be_11 = b_e.reshape(1, 1).astype(f32)
    wv = jnp.stack([jnp.asarray(weekday, f32).reshape(()),
                    jnp.asarray(vehicles, f32).reshape(())]).reshape(1, 2)

    const_spec = lambda shape: pl.BlockSpec(shape, lambda i: (0, 0))

    out = pl.pallas_call(
        _fused_kernel,
        grid=(2 * NB,),
        in_specs=[
            const_spec((N, EMB)),
            const_spec((1, S)),
            const_spec((EMB, P)), const_spec((EMB, P)),
            const_spec((1, P)), const_spec((1, 2 * P)),
            pl.BlockSpec((R, N), lambda i: (jnp.maximum(i - NB, 0), 0)),
            const_spec((P + 2 * N + 3, N)),
            const_spec((1, N)), const_spec((1, 1)), const_spec((1, 2)),
        ],
        out_specs=pl.BlockSpec((R, N), lambda i: (jnp.maximum(i - NB, 0), 0)),
        out_shape=jax.ShapeDtypeStruct((N, N), f32),
        scratch_shapes=[
            pltpu.VMEM((N, EMB), f32),
            pltpu.VMEM((N, P), f32),
            pltpu.VMEM((N, 1), f32),
            pltpu.VMEM((N, 1), f32),
            pltpu.VMEM((1, N), f32),
            pltpu.VMEM((1, N), f32),
            pltpu.VMEM((1, N), f32),
            pltpu.VMEM((1, N), f32),
            pltpu.VMEM((1, N), f32),
            pltpu.VMEM((1, N), f32),
        ],
    )(emb.astype(f32), stops_r, W_l.astype(f32), W_r.astype(f32),
      b_l_row, W_eT, dist.astype(f32), W_c.astype(f32), bc_row, be_11, wv)
    return out
